# baseline (speedup 1.0000x reference)
import functools

import jax
import jax.numpy as jnp
from jax import lax
from jax.experimental import pallas as pl
from jax.experimental.pallas import tpu as pltpu
from jax.experimental.pallas import tpu_sc as plsc

LQ = 2048
LK = 2048
ROWS = 257
MAXREL = 128
D = 32
CPW = 4
MT = 23

_PAIRS = (((0, 0, 8), (2, 0, 8)), ((0, 8, 0), (2, 8, 0)))


def _sc_body(table_hbm, out_hbm, tab_v, buf0, buf1, sem0, sem1):
    cid = lax.axis_index("c")
    sid = lax.axis_index("s")
    wid = sid * 2 + cid

    pltpu.sync_copy(table_hbm, tab_v)
    iota = lax.iota(jnp.int32, 16)

    def drain(sem, n=16):
        for _ in range(n):
            pltpu.make_async_copy(
                buf0.at[0, pl.ds(0, 16)], out_hbm.at[0, 0], sem
            ).wait()

    def build(buf, i0, rlo, mlo):
        def m_body(m_rel, carry):
            col = 127 - i0 + 128 * (mlo + m_rel)
            is_const = (col <= 1792) | (col >= 2175)

            @pl.when(is_const)
            def _const_tile():
                cbase = jnp.where(col <= 1792, 129, 128) * D + 8 * rlo
                vrow = tab_v[pl.ds(cbase, 16)]
                for rr in range(2):
                    for qq in range(8):
                        vec = jnp.broadcast_to(vrow[rr * 8 + qq], (16,))
                        for lc in range(8):
                            buf[rr, m_rel, qq, pl.ds(lc * 16, 16)] = vec

            @pl.when(~is_const)
            def _var_tile():
                for lc in range(8):
                    t = col + lc * 16 + iota
                    c = jnp.clip(t - (LQ - 1), -MAXREL, MAXREL)
                    rb = jnp.where(c < 0, c + ROWS, c) * D
                    for rr in range(2):
                        for qq in range(8):
                            g = plsc.load_gather(tab_v, [rb + (8 * (rlo + rr) + qq)])
                            buf[rr, m_rel, qq, pl.ds(lc * 16, 16)] = g

            return carry

        lax.fori_loop(0, MT, m_body, 0)

    def fire(buf, i0, rlo, mlo, klo, sem):
        for k_rel in range(8):
            k = klo + k_rel
            pltpu.async_copy(
                buf.at[:, pl.ds(15 - k - mlo, 16)],
                out_hbm.at[i0 + 128 * k, pl.ds(rlo, 2)],
                sem,
            )

    for pair_idx, (qa, qb) in enumerate(_PAIRS):
        def pair_body(ci, carry, qa=qa, qb=qb, pair_idx=pair_idx):
            i0 = wid * CPW + ci
            for buf, sem, (rlo, mlo, klo) in ((buf0, sem0, qa), (buf1, sem1, qb)):
                if pair_idx == 0:
                    @pl.when(ci > 0)
                    def _d():
                        drain(sem)
                else:
                    drain(sem)
                build(buf, i0, rlo, mlo)
                fire(buf, i0, rlo, mlo, klo, sem)
            return carry

        lax.fori_loop(0, CPW, pair_body, 0)

    drain(sem0)
    drain(sem1)


_sc_call = functools.partial(
    pl.kernel,
    out_type=jax.ShapeDtypeStruct((LQ, 4, 16, 8, 128), jnp.float32),
    mesh=plsc.VectorSubcoreMesh(core_axis_name="c", subcore_axis_name="s"),
    scratch_types=[
        pltpu.VMEM((ROWS * D,), jnp.float32),
        pltpu.VMEM((2, MT, 8, 128), jnp.float32),
        pltpu.VMEM((2, MT, 8, 128), jnp.float32),
        pltpu.SemaphoreType.DMA,
        pltpu.SemaphoreType.DMA,
    ],
    compiler_params=pltpu.CompilerParams(
        use_tc_tiling_on_sc=False, needs_layout_passes=False
    ),
)(_sc_body)


def kernel(length_q, length_k, embeddings_table):
    del length_q, length_k
    x = _sc_call(embeddings_table.reshape(ROWS * D))
    return x.transpose(0, 2, 4, 1, 3).reshape(LQ, LK, D)
